# baseline (speedup 1.0000x reference)
import functools

import jax
import jax.numpy as jnp
import numpy as np
from jax import lax
from jax.experimental import pallas as pl
from jax.experimental.pallas import tpu as pltpu
from jax.experimental.pallas import tpu_sc as plsc

EPS = 1e-5
B = 16384
BM = 2048
NCH = 2


def _sc_gather(table, idx):
    nb = idx.shape[0]
    info = plsc.get_sparse_core_info()
    nc, ns = info.num_cores, info.num_subcores
    nw = nc * ns
    b_per_w = nb // nw
    n_chunks = b_per_w // 128
    idx2d = idx.reshape(nb // 128, 128)
    d = table.shape[1]

    @functools.partial(
        pl.kernel,
        out_type=jax.ShapeDtypeStruct((nb, d), jnp.float32),
        mesh=plsc.VectorSubcoreMesh(core_axis_name="c", subcore_axis_name="s"),
        scratch_types=[
            pltpu.VMEM((n_chunks, 128), jnp.int32),
            pltpu.VMEM((b_per_w, d), jnp.float32),
            pltpu.SemaphoreType.DMA,
        ],
    )
    def k(table_hbm, idx_hbm, out_hbm, idx_v, rows_v, sem):
        wid = lax.axis_index("s") * nc + lax.axis_index("c")
        base = wid * b_per_w
        pltpu.sync_copy(idx_hbm.at[pl.ds(wid * n_chunks, n_chunks)], idx_v)
        copies = []
        for j in range(n_chunks):
            copies.append(
                pltpu.async_copy(
                    table_hbm.at[idx_v.at[j]],
                    rows_v.at[pl.ds(j * 128, 128)],
                    sem,
                )
            )
        for c in copies:
            c.wait()
        pltpu.sync_copy(rows_v, out_hbm.at[pl.ds(base, b_per_w)])

    return k(table, idx2d)


def _mlp_body(comp_ref, sg_ref, lat_ref, a1_ref, a3_ref, a2_ref, b1_ref,
              w2_ref, b2_ref, w3_ref, b3_ref, w4_ref, b4_ref, out_ref):
    h = (
        jnp.dot(comp_ref[...], a1_ref[...], preferred_element_type=jnp.float32)
        + jnp.dot(sg_ref[...], a3_ref[...], preferred_element_type=jnp.float32)
        + jnp.dot(lat_ref[...], a2_ref[...], preferred_element_type=jnp.float32)
        + b1_ref[...]
    )
    h = jnp.maximum(h, 0.2 * h)
    h = jnp.dot(h, w2_ref[...], preferred_element_type=jnp.float32) + b2_ref[...]
    h = jnp.maximum(h, 0.2 * h)
    h = jnp.dot(h, w3_ref[...], preferred_element_type=jnp.float32) + b3_ref[...]
    h = jnp.maximum(h, 0.2 * h)
    out_ref[...] = (
        jnp.dot(h, w4_ref[...], preferred_element_type=jnp.float32) + b4_ref[...]
    )


def _row_spec(bm, d):
    return pl.BlockSpec((bm, d), lambda i: (i, 0))


def _full_spec(shape):
    return pl.BlockSpec(shape, lambda i: (0, 0))


def _tc_mlp(comp_x, sg_emb, lat8, a1, a3, a2, b1, w2, b2, w3, b3, w4, b4):
    nb = comp_x.shape[0]
    grid = (nb // BM,)
    return pl.pallas_call(
        _mlp_body,
        grid=grid,
        in_specs=[
            _row_spec(BM, 128),
            _row_spec(BM, 128),
            _row_spec(BM, 8),
            _full_spec(a1.shape),
            _full_spec(a3.shape),
            _full_spec(a2.shape),
            _full_spec(b1.shape),
            _full_spec(w2.shape),
            _full_spec(b2.shape),
            _full_spec(w3.shape),
            _full_spec(b3.shape),
            _full_spec(w4.shape),
            _full_spec(b4.shape),
        ],
        out_specs=_row_spec(BM, 1),
        out_shape=jax.ShapeDtypeStruct((nb, 1), jnp.float32),
    )(comp_x, sg_emb, lat8, a1, a3, a2, b1, w2, b2, w3, b3, w4, b4)


def kernel(comp_x, sg_x, lat_x, sg_table, cW1, cb1, cg1, cbeta1, cW2, cb2, cg2,
           cbeta2, lW1, lb1, lg1, lbeta1, lW2, lb2, lg2, lbeta2, pW1, pb1, pg1,
           pbeta1, pW2, pb2, pg2, pbeta2, pW3, pb3, pg3, pbeta3, pW4, pb4):
    s = np.float32(1.0 / np.sqrt(1.0 + EPS))

    a1c, a2c = cg1 * s, cg2 * s
    mc = (cW1.T * a1c[None, :]) @ (cW2.T * a2c[None, :])
    vc = (a1c * cb1 + cbeta1) @ (cW2.T * a2c[None, :]) + (a2c * cb2 + cbeta2)
    a1l, a2l = lg1 * s, lg2 * s
    ml = (lW1.T * a1l[None, :]) @ (lW2.T * a2l[None, :])
    vl = (a1l * lb1 + lbeta1) @ (lW2.T * a2l[None, :]) + (a2l * lb2 + lbeta2)

    p1 = pW1.T
    p1c, p1s, p1l = p1[:256], p1[256:320], p1[320:]
    ap1 = pg1 * s
    a1m = (mc @ p1c) * ap1[None, :]
    a3m = jnp.concatenate(
        [p1s * ap1[None, :], jnp.zeros((64, 512), jnp.float32)], axis=0
    )
    a2m = (ml @ p1l) * ap1[None, :]
    b1v = ap1 * (vc @ p1c + vl @ p1l + pb1) + pbeta1
    a2m = jnp.concatenate([a2m, jnp.zeros((2, 512), jnp.float32)], axis=0)

    ap2 = pg2 * s
    w2 = pW2.T * ap2[None, :]
    b2v = ap2 * pb2 + pbeta2
    ap3 = pg3 * s
    w3 = pW3.T * ap3[None, :]
    b3v = ap3 * pb3 + pbeta3
    w4 = pW4.T
    b4v = pb4

    lat8 = jnp.concatenate([lat_x, jnp.zeros((B, 2), jnp.float32)], axis=1)
    idx = sg_x[:, 0].astype(jnp.int32)
    table_pad = jnp.pad(sg_table, ((0, 2), (0, 64)))

    cb = B // NCH
    outs = []
    for c in range(NCH):
        sl = slice(c * cb, (c + 1) * cb)
        sg_emb = _sc_gather(table_pad, idx[sl])
        outs.append(
            _tc_mlp(
                comp_x[sl], sg_emb, lat8[sl],
                a1m, a3m, a2m, b1v.reshape(1, 512),
                w2, b2v.reshape(1, 256),
                w3, b3v.reshape(1, 128),
                w4, b4v.reshape(1, 1),
            )
        )
    return jnp.concatenate(outs, axis=0) if NCH > 1 else outs[0]


# speedup vs baseline: 1.1934x; 1.1934x over previous
import functools

import jax
import jax.numpy as jnp
import numpy as np
from jax import lax
from jax.experimental import pallas as pl
from jax.experimental.pallas import tpu as pltpu
from jax.experimental.pallas import tpu_sc as plsc

EPS = 1e-5
B = 16384
BM = 2048
_S = np.float32(1.0 / np.sqrt(1.0 + EPS))


def _sc_gather(table, idx):
    nb = idx.shape[0]
    info = plsc.get_sparse_core_info()
    nc, ns = info.num_cores, info.num_subcores
    nw = nc * ns
    b_per_w = nb // nw
    n_chunks = b_per_w // 128
    idx2d = idx.reshape(nb // 128, 128)
    d = table.shape[1]

    @functools.partial(
        pl.kernel,
        out_type=jax.ShapeDtypeStruct((nb, d), jnp.float32),
        mesh=plsc.VectorSubcoreMesh(core_axis_name="c", subcore_axis_name="s"),
        scratch_types=[
            pltpu.VMEM((n_chunks, 128), jnp.int32),
            pltpu.VMEM((b_per_w, d), jnp.float32),
            pltpu.SemaphoreType.DMA,
        ],
    )
    def k(table_hbm, idx_hbm, out_hbm, idx_v, rows_v, sem):
        wid = lax.axis_index("s") * nc + lax.axis_index("c")
        base = wid * b_per_w
        pltpu.sync_copy(idx_hbm.at[pl.ds(wid * n_chunks, n_chunks)], idx_v)
        copies = []
        for j in range(n_chunks):
            copies.append(
                pltpu.async_copy(
                    table_hbm.at[idx_v.at[j]],
                    rows_v.at[pl.ds(j * 128, 128)],
                    sem,
                )
            )
        for c in copies:
            c.wait()
        pltpu.sync_copy(rows_v, out_hbm.at[pl.ds(base, b_per_w)])

    return k(table, idx2d)


def _dgt(a, b):
    return lax.dot_general(a, b, (((1,), (1,)), ((), ())),
                           preferred_element_type=jnp.float32)


def _fold_body(cW1_r, cW2_r, lW1_r, lW2_r, pW1_r,
               cg1c_r, cg2c_r, lg1c_r, lg2c_r, pg1c_r,
               cb1_r, cbeta1_r, cg2_r, cb2_r, cbeta2_r,
               lb1_r, lbeta1_r, lg2_r, lb2_r, lbeta2_r,
               lg1_r, cg1_r, pg1_r, pb1_r, pbeta1_r,
               a1m_o, p2s_o, a2m_o, b1v_o):
    c1 = cW1_r[...] * (cg1c_r[...] * _S)
    c2 = cW2_r[...] * (cg2c_r[...] * _S)
    mc = lax.dot_general(c1, c2, (((0,), (1,)), ((), ())),
                         preferred_element_type=jnp.float32)
    vc = cg1_r[...] * _S * cb1_r[...] + cbeta1_r[...]
    vc = _dgt(vc, c2) + cg2_r[...] * _S * cb2_r[...] + cbeta2_r[...]

    l1 = lW1_r[...] * (lg1c_r[...] * _S)
    l2 = lW2_r[...] * (lg2c_r[...] * _S)
    ml = lax.dot_general(l1, l2, (((0,), (1,)), ((), ())),
                         preferred_element_type=jnp.float32)
    vl = lg1_r[...] * _S * lb1_r[...] + lbeta1_r[...]
    vl = _dgt(vl, l2) + lg2_r[...] * _S * lb2_r[...] + lbeta2_r[...]

    ap1c = pg1c_r[...] * _S
    p1c = pW1_r[:, 0:256] * ap1c
    p2s_o[...] = pW1_r[:, 256:320] * ap1c
    p1l = pW1_r[:, 320:576] * ap1c

    a1m_o[...] = _dgt(mc, p1c)
    a2m_o[...] = _dgt(ml, p1l)
    b1v_o[...] = (
        _dgt(vc, p1c) + _dgt(vl, p1l)
        + pg1_r[...] * _S * pb1_r[...] + pbeta1_r[...]
    )


def _fold(cW1, cW2, lW1p, lW2, pW1,
          cg1, cg2, lg1, lg2, pg1,
          cb1, cbeta1, cb2, cbeta2, lb1, lbeta1, lb2, lbeta2, pb1, pbeta1):
    col = lambda x: x.reshape(-1, 1)
    row = lambda x: x.reshape(1, -1)
    args = (cW1, cW2, lW1p, lW2, pW1,
            col(cg1), col(cg2), col(lg1), col(lg2), col(pg1),
            row(cb1), row(cbeta1), row(cg2), row(cb2), row(cbeta2),
            row(lb1), row(lbeta1), row(lg2), row(lb2), row(lbeta2),
            row(lg1), row(cg1), row(pg1), row(pb1), row(pbeta1))
    return pl.pallas_call(
        _fold_body,
        out_shape=(
            jax.ShapeDtypeStruct((128, 512), jnp.float32),
            jax.ShapeDtypeStruct((512, 64), jnp.float32),
            jax.ShapeDtypeStruct((8, 512), jnp.float32),
            jax.ShapeDtypeStruct((1, 512), jnp.float32),
        ),
    )(*args)


def _mlp_body(comp_ref, sg_ref, lat_ref, a1m_ref, p2s_ref, a2m_ref, b1v_ref,
              pW2_ref, pg2_ref, pb2_ref, pbeta2_ref,
              pW3_ref, pg3_ref, pb3_ref, pbeta3_ref,
              pW4_ref, pb4_ref, out_ref):
    h = (
        jnp.dot(comp_ref[...], a1m_ref[...], preferred_element_type=jnp.float32)
        + _dgt(sg_ref[...][:, :64], p2s_ref[...])
        + jnp.dot(lat_ref[...], a2m_ref[...], preferred_element_type=jnp.float32)
        + b1v_ref[...]
    )
    h = jnp.maximum(h, 0.2 * h)
    ap2 = pg2_ref[...] * _S
    h = _dgt(h, pW2_ref[...]) * ap2 + (ap2 * pb2_ref[...] + pbeta2_ref[...])
    h = jnp.maximum(h, 0.2 * h)
    ap3 = pg3_ref[...] * _S
    h = _dgt(h, pW3_ref[...]) * ap3 + (ap3 * pb3_ref[...] + pbeta3_ref[...])
    h = jnp.maximum(h, 0.2 * h)
    out_ref[...] = (
        jnp.sum(h * pW4_ref[...], axis=1, keepdims=True) + pb4_ref[...]
    )


def _row_spec(bm, d):
    return pl.BlockSpec((bm, d), lambda i: (i, 0))


def _full_spec(shape):
    return pl.BlockSpec(shape, lambda i: (0,) * len(shape))


def _tc_mlp(comp_x, sg_emb, lat8, a1m, p2s, a2m, b1v,
            pW2, pg2, pb2, pbeta2, pW3, pg3, pb3, pbeta3, pW4, pb4):
    nb = comp_x.shape[0]
    smalls = (a1m, p2s, a2m, b1v, pW2, pg2, pb2, pbeta2,
              pW3, pg3, pb3, pbeta3, pW4, pb4)
    return pl.pallas_call(
        _mlp_body,
        grid=(nb // BM,),
        in_specs=[
            _row_spec(BM, 128),
            _row_spec(BM, 128),
            _row_spec(BM, 8),
        ] + [_full_spec(s.shape) for s in smalls],
        out_specs=_row_spec(BM, 1),
        out_shape=jax.ShapeDtypeStruct((nb, 1), jnp.float32),
    )(comp_x, sg_emb, lat8, *smalls)


def kernel(comp_x, sg_x, lat_x, sg_table, cW1, cb1, cg1, cbeta1, cW2, cb2, cg2,
           cbeta2, lW1, lb1, lg1, lbeta1, lW2, lb2, lg2, lbeta2, pW1, pb1, pg1,
           pbeta1, pW2, pb2, pg2, pbeta2, pW3, pb3, pg3, pbeta3, pW4, pb4):
    idx = sg_x[:, 0].astype(jnp.int32)
    table_pad = jnp.pad(sg_table, ((0, 2), (0, 64)))
    sg_emb = _sc_gather(table_pad, idx)

    lat8 = jnp.pad(lat_x, ((0, 0), (0, 2)))
    lW1p = jnp.pad(lW1, ((0, 0), (0, 2)))

    a1m, p2s, a2m, b1v = _fold(
        cW1, cW2, lW1p, lW2, pW1, cg1, cg2, lg1, lg2, pg1,
        cb1, cbeta1, cb2, cbeta2, lb1, lbeta1, lb2, lbeta2, pb1, pbeta1)

    row = lambda x: x.reshape(1, -1)
    return _tc_mlp(
        comp_x, sg_emb, lat8, a1m, p2s, a2m, b1v,
        pW2, row(pg2), row(pb2), row(pbeta2),
        pW3, row(pg3), row(pb3), row(pbeta3),
        pW4.reshape(1, 128), row(pb4),
    )
